# baseline (speedup 1.0000x reference)
import functools

import jax
import jax.numpy as jnp
from jax import lax
from jax.experimental import pallas as pl
from jax.experimental.pallas import tpu as pltpu
from jax.experimental.pallas import tpu_sc as plsc

_N = 128
_D = 768
_H = 768
_E = 64
_K = 2
_EPB = 4


def _moe_body(x_ref, wg_ref, bg_ref, w1_ref, b1_ref, w2_ref, b2_ref,
              out_ref, loss_ref, comb_ref):
    e = pl.program_id(0)

    @pl.when(e == 0)
    def _routing():
        logits = jnp.dot(x_ref[...], wg_ref[...],
                         preferred_element_type=jnp.float32) + bg_ref[...]
        m = jnp.max(logits, axis=-1, keepdims=True)
        p = jnp.exp(logits - m)
        gates = p / jnp.sum(p, axis=-1, keepdims=True)
        col = jax.lax.broadcasted_iota(jnp.int32, (_N, _E), 1)
        i1 = jnp.argmax(gates, axis=-1)
        oh1 = col == i1[:, None]
        g2 = jnp.where(oh1, -jnp.inf, gates)
        i2 = jnp.argmax(g2, axis=-1)
        oh2 = col == i2[:, None]
        sel = oh1 | oh2
        comb_ref[...] = jnp.where(sel, gates, 0.0)
        load = jnp.sum(sel.astype(jnp.float32), axis=0)
        frac = load / jnp.float32(_N * _K)
        loss_ref[...] = jnp.full((1, 1), jnp.sum(frac * frac), jnp.float32)
        out_ref[...] = jnp.zeros_like(out_ref)

    col = jax.lax.broadcasted_iota(jnp.int32, (_N, _E), 1)
    xb = x_ref[...].astype(jnp.bfloat16)
    acc = jnp.zeros((_N, _D), jnp.float32)
    for j in range(_EPB):
        w = jnp.sum(jnp.where(col == e * _EPB + j, comb_ref[...], 0.0),
                    axis=1)
        h = jnp.maximum(
            jnp.dot(xb, w1_ref[j].astype(jnp.bfloat16),
                    preferred_element_type=jnp.float32) + b1_ref[j], 0.0)
        y = jnp.dot(h.astype(jnp.bfloat16), w2_ref[j].astype(jnp.bfloat16),
                    preferred_element_type=jnp.float32) + b2_ref[j]
        acc += w[:, None] * y
    out_ref[...] += acc


def kernel(x, Wg, bg, W1, b1, W2, b2):
    out, loss = pl.pallas_call(
        _moe_body,
        grid=(_E // _EPB,),
        in_specs=[
            pl.BlockSpec((_N, _D), lambda e: (0, 0)),
            pl.BlockSpec((_D, _E), lambda e: (0, 0)),
            pl.BlockSpec((1, _E), lambda e: (0, 0)),
            pl.BlockSpec((_EPB, _D, _H), lambda e: (e, 0, 0)),
            pl.BlockSpec((_EPB, 1, _H), lambda e: (e, 0, 0)),
            pl.BlockSpec((_EPB, _H, _D), lambda e: (e, 0, 0)),
            pl.BlockSpec((_EPB, 1, _D), lambda e: (e, 0, 0)),
        ],
        out_specs=[
            pl.BlockSpec((_N, _D), lambda e: (0, 0)),
            pl.BlockSpec((1, 1), lambda e: (0, 0)),
        ],
        out_shape=[
            jax.ShapeDtypeStruct((_N, _D), jnp.float32),
            jax.ShapeDtypeStruct((1, 1), jnp.float32),
        ],
        scratch_shapes=[pltpu.VMEM((_N, _E), jnp.float32)],
        compiler_params=pltpu.CompilerParams(
            dimension_semantics=("arbitrary",),
        ),
    )(x, Wg, bg.reshape(1, _E), W1, b1.reshape(_E, 1, _H), W2,
      b2.reshape(_E, 1, _D))
    probe = _sc_probe(W1.reshape(-1))
    return out, loss.reshape(()) + jnp.sum(probe) * jnp.float32(1e-40)


_CH = 32768
_NCHUNK = 9


@functools.partial(
    pl.kernel,
    mesh=plsc.VectorSubcoreMesh(core_axis_name="c", subcore_axis_name="s"),
    out_type=jax.ShapeDtypeStruct((16,), jnp.float32),
    scratch_types=[pltpu.VMEM((_CH,), jnp.float32)],
)
def _sc_probe(w1_hbm, out_hbm, buf):
    wid = lax.axis_index("s") * 2 + lax.axis_index("c")
    base = wid * (_CH * _NCHUNK)

    def body(i, carry):
        pltpu.sync_copy(w1_hbm.at[pl.ds(base + i * _CH, _CH)], buf)
        return carry

    lax.fori_loop(0, _NCHUNK, body, 0)

    @pl.when(wid == 0)
    def _():
        pltpu.sync_copy(buf.at[pl.ds(0, 16)], out_hbm)


# speedup vs baseline: 2.3307x; 2.3307x over previous
import functools

import jax
import jax.numpy as jnp
from jax import lax
from jax.experimental import pallas as pl
from jax.experimental.pallas import tpu as pltpu
from jax.experimental.pallas import tpu_sc as plsc

_N = 128
_D = 768
_H = 768
_E = 64
_K = 2
_EPB = 2


def _gating_body(x_ref, wg_ref, bg_ref, gt_ref):
    logits = jnp.dot(x_ref[...], wg_ref[...],
                     preferred_element_type=jnp.float32) + bg_ref[...]
    m = jnp.max(logits, axis=-1, keepdims=True)
    p = jnp.exp(logits - m)
    gt_ref[...] = (p / jnp.sum(p, axis=-1, keepdims=True)).T


def _moe_body(x_ref, wg_ref, bg_ref, w1_ref, b1_ref, w2_ref, b2_ref,
              out_ref, comb_ref):
    e = pl.program_id(0)

    @pl.when(e == 0)
    def _routing():
        logits = jnp.dot(x_ref[...], wg_ref[...],
                         preferred_element_type=jnp.float32) + bg_ref[...]
        m = jnp.max(logits, axis=-1, keepdims=True)
        p = jnp.exp(logits - m)
        gates = p / jnp.sum(p, axis=-1, keepdims=True)
        col = jax.lax.broadcasted_iota(jnp.int32, (_N, _E), 1)
        i1 = jnp.argmax(gates, axis=-1)
        oh1 = col == i1[:, None]
        g2 = jnp.where(oh1, -jnp.inf, gates)
        i2 = jnp.argmax(g2, axis=-1)
        oh2 = col == i2[:, None]
        comb_ref[...] = jnp.where(oh1 | oh2, gates, 0.0)
        out_ref[...] = jnp.zeros_like(out_ref)

    col = jax.lax.broadcasted_iota(jnp.int32, (_N, _E), 1)
    xb = x_ref[...].astype(jnp.bfloat16)
    acc = jnp.zeros((_N, _D), jnp.float32)
    for j in range(_EPB):
        w = jnp.sum(jnp.where(col == e * _EPB + j, comb_ref[...], 0.0),
                    axis=1)
        h = jnp.maximum(
            jnp.dot(xb, w1_ref[j].astype(jnp.bfloat16),
                    preferred_element_type=jnp.float32) + b1_ref[j], 0.0)
        y = jnp.dot(h.astype(jnp.bfloat16), w2_ref[j].astype(jnp.bfloat16),
                    preferred_element_type=jnp.float32) + b2_ref[j]
        acc += w[:, None] * y
    out_ref[...] += acc


@functools.partial(
    pl.kernel,
    mesh=plsc.VectorSubcoreMesh(core_axis_name="c", subcore_axis_name="s"),
    out_type=jax.ShapeDtypeStruct((16,), jnp.float32),
    compiler_params=pltpu.CompilerParams(needs_layout_passes=False),
    scratch_types=[
        pltpu.VMEM((_E, _N), jnp.float32),
        pltpu.VMEM((8, 16), jnp.int32),
        pltpu.VMEM((8, 16), jnp.int32),
        pltpu.VMEM((16,), jnp.float32),
    ],
)
def _sc_loss(gatest_hbm, loss_hbm, gtbuf, i1buf, i2buf, lbuf):
    cid = lax.axis_index("c")
    sid = lax.axis_index("s")

    @pl.when((cid == 0) & (sid == 0))
    def _():
        pltpu.sync_copy(gatest_hbm, gtbuf)
        neg = jnp.full((16,), -3e38, jnp.float32)
        zi = jnp.zeros((16,), jnp.int32)

        for g in range(_N // 16):

            def estep(e2, st):
                m1, m2, i1, i2 = st
                ev = jnp.full((16,), e2, jnp.int32)
                v = gtbuf[e2, pl.ds(16 * g, 16)]
                gt1 = v > m1
                gt2 = v > m2
                i2n = jnp.where(gt1, i1, jnp.where(gt2, ev, i2))
                m2n = jnp.where(gt1, m1, jnp.where(gt2, v, m2))
                return (jnp.where(gt1, v, m1), m2n,
                        jnp.where(gt1, ev, i1), i2n)

            _, _, i1, i2 = lax.fori_loop(0, _E, estep, (neg, neg, zi, zi))
            i1buf[g, :] = i1
            i2buf[g, :] = i2

        def per_expert(e, lacc):
            ev = jnp.full((16,), e, jnp.int32)

            def grp(g, cacc):
                s1 = i1buf[g, :] == ev
                s2 = i2buf[g, :] == ev
                return (cacc + plsc.all_reduce_population_count(s1)
                        + plsc.all_reduce_population_count(s2))

            cnt = lax.fori_loop(0, _N // 16, grp, zi)
            f = cnt.astype(jnp.float32) / jnp.float32(_N * _K)
            return lacc + f * f

        lbuf[...] = lax.fori_loop(0, _E, per_expert,
                                  jnp.zeros((16,), jnp.float32))
        pltpu.sync_copy(lbuf, loss_hbm)


def kernel(x, Wg, bg, W1, b1, W2, b2):
    gatest = pl.pallas_call(
        _gating_body,
        in_specs=[
            pl.BlockSpec((_N, _D), lambda: (0, 0)),
            pl.BlockSpec((_D, _E), lambda: (0, 0)),
            pl.BlockSpec((1, _E), lambda: (0, 0)),
        ],
        out_specs=pl.BlockSpec((_E, _N), lambda: (0, 0)),
        out_shape=jax.ShapeDtypeStruct((_E, _N), jnp.float32),
    )(x, Wg, bg.reshape(1, _E))

    loss_vec = _sc_loss(gatest)

    out = pl.pallas_call(
        _moe_body,
        grid=(_E // _EPB,),
        in_specs=[
            pl.BlockSpec((_N, _D), lambda e: (0, 0)),
            pl.BlockSpec((_D, _E), lambda e: (0, 0)),
            pl.BlockSpec((1, _E), lambda e: (0, 0)),
            pl.BlockSpec((_EPB, _D, _H), lambda e: (e, 0, 0)),
            pl.BlockSpec((_EPB, 1, _H), lambda e: (e, 0, 0)),
            pl.BlockSpec((_EPB, _H, _D), lambda e: (e, 0, 0)),
            pl.BlockSpec((_EPB, 1, _D), lambda e: (e, 0, 0)),
        ],
        out_specs=pl.BlockSpec((_N, _D), lambda e: (0, 0)),
        out_shape=jax.ShapeDtypeStruct((_N, _D), jnp.float32),
        scratch_shapes=[pltpu.VMEM((_N, _E), jnp.float32)],
        compiler_params=pltpu.CompilerParams(
            dimension_semantics=("arbitrary",),
        ),
    )(x, Wg, bg.reshape(1, _E), W1, b1.reshape(_E, 1, _H), W2,
      b2.reshape(_E, 1, _D))
    return out, loss_vec[0]
